# baseline (speedup 1.0000x reference)
import functools

import jax
import jax.numpy as jnp
from jax import lax
from jax.experimental import pallas as pl
from jax.experimental.pallas import tpu as pltpu
from jax.experimental.pallas import tpu_sc as plsc

D = 64
NC = 2
NS = 16
NW = NC * NS
L = 16


def _sqrt16(x):
    x = jnp.maximum(x, jnp.float32(1e-30))
    i = plsc.bitcast(x, jnp.int32)
    r = plsc.bitcast(jnp.int32(0x5F3759DF) - lax.shift_right_logical(i, 1),
                     jnp.float32)
    for _ in range(3):
        r = r * (jnp.float32(1.5) - jnp.float32(0.5) * x * r * r)
    return x * r


def kernel(head_entities, relations, tail_entities, entity_embeddings,
           relation_embeddings, entity_normal_vectors,
           relation_normal_vectors):
    B = head_entities.shape[0]
    rows_per_worker = B // NW
    CHUNK = 128
    NCHUNK = rows_per_worker // CHUNK

    h_idx = head_entities.reshape(NW, NCHUNK, CHUNK)
    r_idx = relations.reshape(NW, NCHUNK, CHUNK)
    t_idx = tail_entities.reshape(NW, NCHUNK, CHUNK)

    mesh = plsc.VectorSubcoreMesh(core_axis_name="c", subcore_axis_name="s",
                                  num_cores=NC, num_subcores=NS)

    @functools.partial(
        pl.kernel,
        out_type=jax.ShapeDtypeStruct((NW, NCHUNK, CHUNK), jnp.float32),
        mesh=mesh,
        compiler_params=pltpu.CompilerParams(needs_layout_passes=False,
                                             use_tc_tiling_on_sc=False),
        scratch_types=[
            pltpu.VMEM((NCHUNK, CHUNK), jnp.int32),
            pltpu.VMEM((NCHUNK, CHUNK), jnp.int32),
            pltpu.VMEM((NCHUNK, CHUNK), jnp.int32),
            pltpu.VMEM((CHUNK, D), jnp.float32),
            pltpu.VMEM((CHUNK, D), jnp.float32),
            pltpu.VMEM((CHUNK, D), jnp.float32),
            pltpu.VMEM((CHUNK, D), jnp.float32),
            pltpu.VMEM((CHUNK, D), jnp.float32),
            pltpu.VMEM((CHUNK, D), jnp.float32),
            pltpu.VMEM((CHUNK,), jnp.float32),
            pltpu.SemaphoreType.DMA,
        ],
    )
    def run(h_hbm, r_hbm, t_hbm, ee_hbm, re_hbm, en_hbm, rn_hbm, out_hbm,
            hidx_v, ridx_v, tidx_v, he_v, hn_v, te_v, tn_v, rre_v, rrn_v,
            sc_v, sem):
        wid = lax.axis_index("s") * NC + lax.axis_index("c")
        pltpu.sync_copy(h_hbm.at[wid], hidx_v)
        pltpu.sync_copy(r_hbm.at[wid], ridx_v)
        pltpu.sync_copy(t_hbm.at[wid], tidx_v)
        iota16 = lax.iota(jnp.int32, L)

        for c in range(NCHUNK):
            descs = [
                pltpu.async_copy(ee_hbm.at[hidx_v.at[c]], he_v, sem),
                pltpu.async_copy(en_hbm.at[hidx_v.at[c]], hn_v, sem),
                pltpu.async_copy(ee_hbm.at[tidx_v.at[c]], te_v, sem),
                pltpu.async_copy(en_hbm.at[tidx_v.at[c]], tn_v, sem),
                pltpu.async_copy(re_hbm.at[ridx_v.at[c]], rre_v, sem),
                pltpu.async_copy(rn_hbm.at[ridx_v.at[c]], rrn_v, sem),
            ]
            for dsc in descs:
                dsc.wait()

            @pl.loop(0, CHUNK // L)
            def _group(g):
                acc_ss = jnp.zeros((L,), jnp.float32)
                for k in range(L):
                    row = g * L + k
                    he = [he_v[row, pl.ds(j * L, L)] for j in range(D // L)]
                    hn = [hn_v[row, pl.ds(j * L, L)] for j in range(D // L)]
                    te = [te_v[row, pl.ds(j * L, L)] for j in range(D // L)]
                    tn = [tn_v[row, pl.ds(j * L, L)] for j in range(D // L)]
                    re = [rre_v[row, pl.ds(j * L, L)] for j in range(D // L)]
                    rn = [rrn_v[row, pl.ds(j * L, L)] for j in range(D // L)]
                    ph = he[0] * hn[0]
                    pt = te[0] * tn[0]
                    pr = re[0] * rn[0]
                    for j in range(1, D // L):
                        ph = ph + he[j] * hn[j]
                        pt = pt + te[j] * tn[j]
                        pr = pr + re[j] * rn[j]
                    sh = jnp.sum(ph)
                    st = jnp.sum(pt)
                    sr = jnp.sum(pr)
                    q = None
                    for j in range(D // L):
                        dj = (he[j] - sh * hn[j]) + (re[j] - sr * rn[j]) \
                            - (te[j] - st * tn[j])
                        q = dj * dj if q is None else q + dj * dj
                    ss = jnp.sum(q)
                    acc_ss = jnp.where(iota16 == k, ss, acc_ss)
                sc_v[pl.ds(g * L, L)] = _sqrt16(acc_ss)

            pltpu.sync_copy(sc_v, out_hbm.at[wid, c])

    out = run(h_idx, r_idx, t_idx, entity_embeddings, relation_embeddings,
              entity_normal_vectors, relation_normal_vectors)
    return out.reshape(B)
